# baseline (speedup 1.0000x reference)
import functools
import math

import jax
import jax.numpy as jnp
from jax.experimental import pallas as pl
from jax.experimental.pallas import tpu as pltpu

KEEP_RATIO = 0.5
MIN_TOKENS = 1


def _score_body(aw_ref, scores_ref, acc_ref):
    h = pl.program_id(1)
    acc_ref[h, :] = jnp.sum(aw_ref[0, 0], axis=0)

    @pl.when(h == pl.num_programs(1) - 1)
    def _():
        scores_ref[0, 0, :] = jnp.mean(acc_ref[...], axis=0)


def _prune_body(k, Tc, scores_ref, hs_ref, out_ref, mask_ref, keep_ref):
    t = pl.program_id(1)

    @pl.when(t == 0)
    def _():
        s = scores_ref[0, 0, :]
        T = s.shape[0]
        s_i = s[:, None]
        s_j = s[None, :]
        i_idx = jax.lax.broadcasted_iota(jnp.int32, (T, T), 0)
        j_idx = jax.lax.broadcasted_iota(jnp.int32, (T, T), 1)
        beats = (s_j > s_i) | ((s_j == s_i) & (j_idx < i_idx))
        rank = jnp.sum(beats.astype(jnp.int32), axis=1)
        pos = jax.lax.broadcasted_iota(jnp.int32, (T,), 0)
        keep = (rank < k) | (pos == 0)
        keep_ref[0, :] = keep.astype(jnp.float32)
        mask_ref[0, 0, :] = keep.astype(jnp.int32)

    out_ref[0] = hs_ref[0] * keep_ref[0, pl.ds(t * Tc, Tc)][:, None]


@jax.jit
def kernel(hidden_states, attention_weights):
    B, T, D = hidden_states.shape
    _, H, _, _ = attention_weights.shape
    k = min(max(MIN_TOKENS, math.ceil(KEEP_RATIO * T)), T)
    TS = 4 if T % 4 == 0 else 1
    Tc = T // TS

    scores = pl.pallas_call(
        _score_body,
        grid=(B, H),
        in_specs=[pl.BlockSpec((1, 1, T, T), lambda b, h: (b, h, 0, 0))],
        out_specs=pl.BlockSpec((1, 1, T), lambda b, h: (b, 0, 0)),
        out_shape=jax.ShapeDtypeStruct((B, 1, T), jnp.float32),
        scratch_shapes=[pltpu.VMEM((H, T), jnp.float32)],
        compiler_params=pltpu.CompilerParams(
            dimension_semantics=("arbitrary", "arbitrary"),
        ),
    )(attention_weights)

    pruned, mask_i32 = pl.pallas_call(
        functools.partial(_prune_body, k, Tc),
        grid=(B, TS),
        in_specs=[
            pl.BlockSpec((1, 1, T), lambda b, t: (b, 0, 0)),
            pl.BlockSpec((1, Tc, D), lambda b, t: (b, t, 0)),
        ],
        out_specs=[
            pl.BlockSpec((1, Tc, D), lambda b, t: (b, t, 0)),
            pl.BlockSpec((1, 1, T), lambda b, t: (b, 0, 0)),
        ],
        out_shape=[
            jax.ShapeDtypeStruct((B, T, D), hidden_states.dtype),
            jax.ShapeDtypeStruct((B, 1, T), jnp.int32),
        ],
        scratch_shapes=[pltpu.VMEM((8, T), jnp.float32)],
        compiler_params=pltpu.CompilerParams(
            dimension_semantics=("arbitrary", "arbitrary"),
        ),
    )(scores, hidden_states)

    return (pruned, mask_i32.reshape(B, T).astype(bool))


# speedup vs baseline: 1.0093x; 1.0093x over previous
import functools
import math

import jax
import jax.numpy as jnp
from jax.experimental import pallas as pl
from jax.experimental.pallas import tpu as pltpu

KEEP_RATIO = 0.5
MIN_TOKENS = 1


def _fused_body(k, H, Tc, aw_ref, hs_ref, out_ref, mask_ref, acc_ref, keep_ref):
    h = pl.program_id(1)

    @pl.when(h < H)
    def _():
        acc_ref[h, :] = jnp.sum(aw_ref[0, 0], axis=0)

    @pl.when(h == H)
    def _():
        s = jnp.mean(acc_ref[...], axis=0)
        T = s.shape[0]
        s_i = s[:, None]
        s_j = s[None, :]
        i_idx = jax.lax.broadcasted_iota(jnp.int32, (T, T), 0)
        j_idx = jax.lax.broadcasted_iota(jnp.int32, (T, T), 1)
        beats = (s_j > s_i) | ((s_j == s_i) & (j_idx < i_idx))
        rank = jnp.sum(beats.astype(jnp.int32), axis=1)
        pos = jax.lax.broadcasted_iota(jnp.int32, (T,), 0)
        keep = (rank < k) | (pos == 0)
        keep_ref[0, :] = keep.astype(jnp.float32)
        mask_ref[0, 0, :] = keep.astype(jnp.int32)

    @pl.when(h >= H)
    def _():
        t = h - H
        out_ref[0] = hs_ref[0] * keep_ref[0, pl.ds(t * Tc, Tc)][:, None]


@jax.jit
def kernel(hidden_states, attention_weights):
    B, T, D = hidden_states.shape
    _, H, _, _ = attention_weights.shape
    k = min(max(MIN_TOKENS, math.ceil(KEEP_RATIO * T)), T)
    TS = 2 if T % 2 == 0 else 1
    Tc = T // TS

    pruned, mask_i32 = pl.pallas_call(
        functools.partial(_fused_body, k, H, Tc),
        grid=(B, H + TS),
        in_specs=[
            pl.BlockSpec(
                (1, 1, T, T), lambda b, h: (b, jnp.minimum(h, H - 1), 0, 0)
            ),
            pl.BlockSpec(
                (1, Tc, D), lambda b, h: (b, jnp.clip(h - H, 0, TS - 1), 0)
            ),
        ],
        out_specs=[
            pl.BlockSpec(
                (1, Tc, D), lambda b, h: (b, jnp.clip(h - H, 0, TS - 1), 0)
            ),
            pl.BlockSpec((1, 1, T), lambda b, h: (b, 0, 0)),
        ],
        out_shape=[
            jax.ShapeDtypeStruct((B, T, D), hidden_states.dtype),
            jax.ShapeDtypeStruct((B, 1, T), jnp.int32),
        ],
        scratch_shapes=[
            pltpu.VMEM((H, T), jnp.float32),
            pltpu.VMEM((8, T), jnp.float32),
        ],
        compiler_params=pltpu.CompilerParams(
            dimension_semantics=("arbitrary", "arbitrary"),
        ),
    )(attention_weights, hidden_states)

    return (pruned, mask_i32.reshape(B, T).astype(bool))


# speedup vs baseline: 1.0320x; 1.0225x over previous
import functools
import math

import jax
import jax.numpy as jnp
from jax.experimental import pallas as pl
from jax.experimental.pallas import tpu as pltpu

KEEP_RATIO = 0.5
MIN_TOKENS = 1


def _score_mask_body(k, aw_ref, mask_ref, acc_ref):
    h = pl.program_id(1)
    acc_ref[h, :] = jnp.sum(aw_ref[0, 0], axis=0)

    @pl.when(h == pl.num_programs(1) - 1)
    def _():
        s = jnp.mean(acc_ref[...], axis=0)
        T = s.shape[0]
        s_i = s[:, None]
        s_j = s[None, :]
        i_idx = jax.lax.broadcasted_iota(jnp.int32, (T, T), 0)
        j_idx = jax.lax.broadcasted_iota(jnp.int32, (T, T), 1)
        beats = (s_j > s_i) | ((s_j == s_i) & (j_idx < i_idx))
        rank = jnp.sum(beats.astype(jnp.int32), axis=1)
        pos = jax.lax.broadcasted_iota(jnp.int32, (T,), 0)
        keep = (rank < k) | (pos == 0)
        mask_ref[0, 0, :] = keep.astype(jnp.int32)


def _prune_body(mask_ref, hs_ref, out_ref):
    out_ref[0] = hs_ref[0] * mask_ref[0, 0, :].astype(out_ref.dtype)[:, None]


@jax.jit
def kernel(hidden_states, attention_weights):
    B, T, D = hidden_states.shape
    _, H, _, _ = attention_weights.shape
    k = min(max(MIN_TOKENS, math.ceil(KEEP_RATIO * T)), T)

    mask_i32 = pl.pallas_call(
        functools.partial(_score_mask_body, k),
        grid=(B, H),
        in_specs=[pl.BlockSpec((1, 1, T, T), lambda b, h: (b, h, 0, 0))],
        out_specs=pl.BlockSpec((1, 1, T), lambda b, h: (b, 0, 0)),
        out_shape=jax.ShapeDtypeStruct((B, 1, T), jnp.int32),
        scratch_shapes=[pltpu.VMEM((H, T), jnp.float32)],
        compiler_params=pltpu.CompilerParams(
            dimension_semantics=("arbitrary", "arbitrary"),
        ),
    )(attention_weights)

    pruned = pl.pallas_call(
        _prune_body,
        grid=(B,),
        in_specs=[
            pl.BlockSpec((1, 1, T), lambda b: (b, 0, 0)),
            pl.BlockSpec((1, T, D), lambda b: (b, 0, 0)),
        ],
        out_specs=pl.BlockSpec((1, T, D), lambda b: (b, 0, 0)),
        out_shape=jax.ShapeDtypeStruct((B, T, D), hidden_states.dtype),
    )(mask_i32, hidden_states)

    return (pruned, mask_i32.reshape(B, T).astype(bool))
